# baseline (speedup 1.0000x reference)
import jax
import jax.numpy as jnp
from jax.experimental import pallas as pl

_H = 4
_D = 128
_SLOPE = 0.2
_B = 16


def _lrelu(x):
    return jnp.where(x >= 0, x, x * _SLOPE)


def _gat_kernel(a_ref, w_ref, asrc_ref, adst_ref, bias_ref, out_ref):
    n = a_ref.shape[-1]
    wt_ref = w_ref[...].T
    xt = jnp.concatenate([a_ref[b] for b in range(_B)], axis=1)
    xtb = xt.astype(jnp.bfloat16)
    vsrc = jnp.concatenate([
        jnp.dot(asrc_ref[h:h + 1, :], wt_ref[h * _D:(h + 1) * _D, :],
                preferred_element_type=jnp.float32)
        for h in range(_H)], axis=0)
    vdst = jnp.concatenate([
        jnp.dot(adst_ref[h:h + 1, :], wt_ref[h * _D:(h + 1) * _D, :],
                preferred_element_type=jnp.float32)
        for h in range(_H)], axis=0)
    a_src_all = jnp.dot(vsrc, xt, preferred_element_type=jnp.float32)
    a_dst_all = jnp.dot(vdst, xt, preferred_element_type=jnp.float32)
    ht = jnp.dot(wt_ref.astype(jnp.bfloat16), xtb,
                 preferred_element_type=jnp.float32).astype(jnp.bfloat16)
    u1_all = jnp.exp(a_src_all).astype(jnp.bfloat16).T
    u2_all = jnp.exp(a_src_all * _SLOPE).astype(jnp.bfloat16).T
    eye = jnp.where(
        jax.lax.broadcasted_iota(jnp.int32, (n, n), 0)
        == jax.lax.broadcasted_iota(jnp.int32, (n, n), 1),
        1.0, 0.0)
    for b in range(_B):
        acc = None
        for hd in range(_H):
            a_src = a_src_all[hd:hd + 1, b * n:(b + 1) * n]
            a_dst = a_dst_all[hd:hd + 1, b * n:(b + 1) * n]
            m = _lrelu(jnp.max(a_src, axis=1, keepdims=True) + a_dst)
            w1 = jnp.exp(a_dst - m).astype(jnp.bfloat16)
            w2 = jnp.exp(a_dst * _SLOPE - m).astype(jnp.bfloat16)
            u1 = u1_all[b * n:(b + 1) * n, hd:hd + 1]
            u2 = u2_all[b * n:(b + 1) * n, hd:hd + 1]
            p = jnp.maximum(u1 * w1, u2 * w2)
            s = jnp.sum(p.astype(jnp.float32), axis=0, keepdims=True)
            r = 1.0 / (s + 1e-16)
            o = jnp.dot(ht[hd * _D:(hd + 1) * _D, b * n:(b + 1) * n], p,
                        preferred_element_type=jnp.float32) * r
            acc = o if acc is None else acc + o
        out_ref[b] = _lrelu(acc * (1.0 / _H) + bias_ref[...]) + eye


def kernel(A, W, att_src, att_dst, bias):
    T, _, N = A.shape
    bias_col = bias.reshape(-1, 1)
    return pl.pallas_call(
        _gat_kernel,
        grid=(T // _B,),
        in_specs=[
            pl.BlockSpec((_B, N, N), lambda t: (t, 0, 0)),
            pl.BlockSpec(W.shape, lambda t: (0, 0)),
            pl.BlockSpec(att_src.shape, lambda t: (0, 0)),
            pl.BlockSpec(att_dst.shape, lambda t: (0, 0)),
            pl.BlockSpec(bias_col.shape, lambda t: (0, 0)),
        ],
        out_specs=pl.BlockSpec((_B, N, N), lambda t: (t, 0, 0)),
        out_shape=jax.ShapeDtypeStruct(A.shape, A.dtype),
    )(A, W, att_src, att_dst, bias_col)


# speedup vs baseline: 1.0196x; 1.0196x over previous
import jax
import jax.numpy as jnp
from jax.experimental import pallas as pl

_H = 4
_D = 128
_SLOPE = 0.2
_B = 32


def _lrelu(x):
    return jnp.where(x >= 0, x, x * _SLOPE)


def _gat_kernel(a_ref, w_ref, asrc_ref, adst_ref, bias_ref, out_ref):
    n = a_ref.shape[-1]
    wt_ref = w_ref[...].T
    xt = jnp.concatenate([a_ref[b] for b in range(_B)], axis=1)
    xtb = xt.astype(jnp.bfloat16)
    vsrc = jnp.concatenate([
        jnp.dot(asrc_ref[h:h + 1, :], wt_ref[h * _D:(h + 1) * _D, :],
                preferred_element_type=jnp.float32)
        for h in range(_H)], axis=0)
    vdst = jnp.concatenate([
        jnp.dot(adst_ref[h:h + 1, :], wt_ref[h * _D:(h + 1) * _D, :],
                preferred_element_type=jnp.float32)
        for h in range(_H)], axis=0)
    a_src_all = jnp.dot(vsrc, xt, preferred_element_type=jnp.float32)
    a_dst_all = jnp.dot(vdst, xt, preferred_element_type=jnp.float32)
    ht = jnp.dot(wt_ref.astype(jnp.bfloat16), xtb,
                 preferred_element_type=jnp.float32).astype(jnp.bfloat16)
    u1_all = jnp.exp(a_src_all).astype(jnp.bfloat16).T
    u2_all = jnp.exp(a_src_all * _SLOPE).astype(jnp.bfloat16).T
    ones_row = jnp.ones((1, n), dtype=jnp.bfloat16)
    eye = jnp.where(
        jax.lax.broadcasted_iota(jnp.int32, (n, n), 0)
        == jax.lax.broadcasted_iota(jnp.int32, (n, n), 1),
        1.0, 0.0)
    for b in range(_B):
        acc = None
        for hd in range(_H):
            a_src = a_src_all[hd:hd + 1, b * n:(b + 1) * n]
            a_dst = a_dst_all[hd:hd + 1, b * n:(b + 1) * n]
            m = _lrelu(jnp.max(a_src, axis=1, keepdims=True) + a_dst)
            w1 = jnp.exp(a_dst - m).astype(jnp.bfloat16)
            w2 = jnp.exp(a_dst * _SLOPE - m).astype(jnp.bfloat16)
            u1 = u1_all[b * n:(b + 1) * n, hd:hd + 1]
            u2 = u2_all[b * n:(b + 1) * n, hd:hd + 1]
            p = jnp.maximum(u1 * w1, u2 * w2)
            s = jnp.dot(ones_row, p, preferred_element_type=jnp.float32)
            r = 1.0 / (s + 1e-16)
            o = jnp.dot(ht[hd * _D:(hd + 1) * _D, b * n:(b + 1) * n], p,
                        preferred_element_type=jnp.float32) * r
            acc = o if acc is None else acc + o
        out_ref[b] = _lrelu(acc * (1.0 / _H) + bias_ref[...]) + eye


def kernel(A, W, att_src, att_dst, bias):
    T, _, N = A.shape
    bias_col = bias.reshape(-1, 1)
    return pl.pallas_call(
        _gat_kernel,
        grid=(T // _B,),
        in_specs=[
            pl.BlockSpec((_B, N, N), lambda t: (t, 0, 0)),
            pl.BlockSpec(W.shape, lambda t: (0, 0)),
            pl.BlockSpec(att_src.shape, lambda t: (0, 0)),
            pl.BlockSpec(att_dst.shape, lambda t: (0, 0)),
            pl.BlockSpec(bias_col.shape, lambda t: (0, 0)),
        ],
        out_specs=pl.BlockSpec((_B, N, N), lambda t: (t, 0, 0)),
        out_shape=jax.ShapeDtypeStruct(A.shape, A.dtype),
    )(A, W, att_src, att_dst, bias_col)


# speedup vs baseline: 1.0756x; 1.0549x over previous
import jax
import jax.numpy as jnp
from jax.experimental import pallas as pl

_H = 4
_D = 128
_SLOPE = 0.2
_B = 32


def _lrelu(x):
    return jnp.where(x >= 0, x, x * _SLOPE)


def _gat_kernel(a_ref, w_ref, asrc_ref, adst_ref, bias_ref, out_ref):
    n = a_ref.shape[-1]
    wt_ref = w_ref[...].T
    xt = jnp.concatenate([a_ref[b] for b in range(_B)], axis=1)
    xtb = xt.astype(jnp.bfloat16)
    vsrc = jnp.concatenate([
        jnp.dot(asrc_ref[h:h + 1, :], wt_ref[h * _D:(h + 1) * _D, :],
                preferred_element_type=jnp.float32)
        for h in range(_H)], axis=0)
    vdst = jnp.concatenate([
        jnp.dot(adst_ref[h:h + 1, :], wt_ref[h * _D:(h + 1) * _D, :],
                preferred_element_type=jnp.float32)
        for h in range(_H)], axis=0)
    a_src_all = jnp.dot(vsrc, xt, preferred_element_type=jnp.float32)
    a_dst_all = jnp.dot(vdst, xt, preferred_element_type=jnp.float32)
    ht = jnp.dot(wt_ref.astype(jnp.bfloat16), xtb,
                 preferred_element_type=jnp.float32).astype(jnp.bfloat16)
    u1_all = jnp.exp(a_src_all).astype(jnp.bfloat16).T
    u2_all = jnp.exp(a_src_all * _SLOPE).astype(jnp.bfloat16).T
    ones_row = jnp.ones((1, n), dtype=jnp.bfloat16)
    bias_full = jnp.broadcast_to(bias_ref[...], (a_ref.shape[1], n))
    eye = jnp.where(
        jax.lax.broadcasted_iota(jnp.int32, (n, n), 0)
        == jax.lax.broadcasted_iota(jnp.int32, (n, n), 1),
        1.0, 0.0)
    for b in range(_B):
        acc = None
        for hd in range(_H):
            a_src = a_src_all[hd:hd + 1, b * n:(b + 1) * n]
            a_dst = a_dst_all[hd:hd + 1, b * n:(b + 1) * n]
            m = _lrelu(jnp.max(a_src, axis=1, keepdims=True) + a_dst)
            w1 = jnp.exp(a_dst - m).astype(jnp.bfloat16)
            w2 = jnp.exp(a_dst * _SLOPE - m).astype(jnp.bfloat16)
            u1 = u1_all[b * n:(b + 1) * n, hd:hd + 1]
            u2 = u2_all[b * n:(b + 1) * n, hd:hd + 1]
            p = jnp.maximum(u1 * w1, u2 * w2)
            s = jnp.dot(ones_row, p, preferred_element_type=jnp.float32)
            r = (1.0 / _H) / (s + 1e-16)
            o = jnp.dot(ht[hd * _D:(hd + 1) * _D, b * n:(b + 1) * n], p,
                        preferred_element_type=jnp.float32) * r
            acc = o if acc is None else acc + o
        out_ref[b] = _lrelu(acc + bias_full) + eye


def kernel(A, W, att_src, att_dst, bias):
    T, _, N = A.shape
    bias_col = bias.reshape(-1, 1)
    return pl.pallas_call(
        _gat_kernel,
        grid=(T // _B,),
        in_specs=[
            pl.BlockSpec((_B, N, N), lambda t: (t, 0, 0)),
            pl.BlockSpec(W.shape, lambda t: (0, 0)),
            pl.BlockSpec(att_src.shape, lambda t: (0, 0)),
            pl.BlockSpec(att_dst.shape, lambda t: (0, 0)),
            pl.BlockSpec(bias_col.shape, lambda t: (0, 0)),
        ],
        out_specs=pl.BlockSpec((_B, N, N), lambda t: (t, 0, 0)),
        out_shape=jax.ShapeDtypeStruct(A.shape, A.dtype),
    )(A, W, att_src, att_dst, bias_col)
